# Initial kernel scaffold; baseline (speedup 1.0000x reference)
import functools

import jax
import jax.numpy as jnp
from jax import lax
from jax.experimental import pallas as pl
from jax.experimental.pallas import tpu as pltpu
from jax.experimental.pallas import tpu_sc as plsc

N = 10000
E = 320000
IN_DIM = 128
HID = 64
OUT = 256
EPS = 1e-5

NC = 2
NS = 16
NW = NC * NS
CH = 80
NCH_W = E // (NW * CH)
NCH_S = E // (NS * CH)
N_PAD = 10016
RPT = N_PAD // NS

_MESH = plsc.VectorSubcoreMesh(
    core_axis_name="c", subcore_axis_name="s", num_cores=NC, num_subcores=NS)


@functools.partial(
    pl.kernel,
    out_type=jax.ShapeDtypeStruct((NC, N_PAD, 8), jnp.float32),
    mesh=_MESH,
    scratch_types=[
        pltpu.VMEM((NCH_W, CH), jnp.int32),
        pltpu.VMEM((CH, 8), jnp.float32),
        pltpu.VMEM_SHARED((N_PAD, 8), jnp.float32),
        pltpu.SemaphoreType.DMA,
    ],
)
def _deg_kernel(dst_hbm, ones_hbm, zeros_hbm, out_hbm, dst_v, ones_v, acc_sh, sem):
    c = lax.axis_index("c")
    s = lax.axis_index("s")
    wid = s * NC + c
    r0 = s * RPT
    pltpu.sync_copy(dst_hbm.at[wid], dst_v)
    pltpu.sync_copy(ones_hbm, ones_v)
    pltpu.sync_copy(zeros_hbm.at[pl.ds(r0, RPT)], acc_sh.at[pl.ds(r0, RPT)])
    plsc.subcore_barrier()

    def body(j, carry):
        pltpu.sync_copy(ones_v, acc_sh.at[dst_v.at[j]], add=True)
        return carry

    lax.fori_loop(0, NCH_W, body, 0)
    plsc.subcore_barrier()
    pltpu.sync_copy(acc_sh.at[pl.ds(r0, RPT)], out_hbm.at[c, pl.ds(r0, RPT)])


@functools.partial(
    pl.kernel,
    out_type=jax.ShapeDtypeStruct((NC, N_PAD, HID), jnp.float32),
    mesh=_MESH,
    scratch_types=[
        pltpu.VMEM((NCH_W, CH), jnp.int32),
        pltpu.VMEM((NCH_W, CH), jnp.int32),
        pltpu.VMEM((CH, HID), jnp.float32),
        pltpu.VMEM_SHARED((N_PAD, HID), jnp.float32),
        pltpu.SemaphoreType.DMA,
    ],
)
def _agg64_kernel(table_hbm, src_hbm, dst_hbm, zeros_hbm, out_hbm,
                  src_v, dst_v, rows_v, acc_sh, sem):
    c = lax.axis_index("c")
    s = lax.axis_index("s")
    wid = s * NC + c
    r0 = s * RPT
    pltpu.sync_copy(src_hbm.at[wid], src_v)
    pltpu.sync_copy(dst_hbm.at[wid], dst_v)
    pltpu.sync_copy(zeros_hbm.at[pl.ds(r0, RPT)], acc_sh.at[pl.ds(r0, RPT)])
    plsc.subcore_barrier()

    def body(j, carry):
        pltpu.async_copy(table_hbm.at[src_v.at[j]], rows_v, sem).wait()
        pltpu.sync_copy(rows_v, acc_sh.at[dst_v.at[j]], add=True)
        return carry

    lax.fori_loop(0, NCH_W, body, 0)
    plsc.subcore_barrier()
    pltpu.sync_copy(acc_sh.at[pl.ds(r0, RPT)], out_hbm.at[c, pl.ds(r0, RPT)])


@functools.partial(
    pl.kernel,
    out_type=jax.ShapeDtypeStruct((NC, N_PAD, 128), jnp.float32),
    mesh=_MESH,
    scratch_types=[
        pltpu.VMEM((NCH_S, CH), jnp.int32),
        pltpu.VMEM((NCH_S, CH), jnp.int32),
        pltpu.VMEM((CH, 128), jnp.float32),
        pltpu.VMEM_SHARED((N_PAD, 128), jnp.float32),
        pltpu.SemaphoreType.DMA,
    ],
)
def _agg128_kernel(table_hbm, src_hbm, dst_hbm, zeros_hbm, out_hbm,
                   src_v, dst_v, rows_v, acc_sh, sem):
    c = lax.axis_index("c")
    s = lax.axis_index("s")
    r0 = s * RPT
    pltpu.sync_copy(src_hbm.at[s], src_v)
    pltpu.sync_copy(dst_hbm.at[s], dst_v)
    pltpu.sync_copy(zeros_hbm.at[pl.ds(r0, RPT)], acc_sh.at[pl.ds(r0, RPT)])
    plsc.subcore_barrier()

    def body(j, carry):
        pltpu.async_copy(table_hbm.at[c].at[src_v.at[j]], rows_v, sem).wait()
        pltpu.sync_copy(rows_v, acc_sh.at[dst_v.at[j]], add=True)
        return carry

    lax.fori_loop(0, NCH_S, body, 0)
    plsc.subcore_barrier()
    pltpu.sync_copy(acc_sh.at[pl.ds(r0, RPT)], out_hbm.at[c, pl.ds(r0, RPT)])


@functools.partial(
    pl.kernel,
    out_type=jax.ShapeDtypeStruct((E,), jnp.float32),
    mesh=_MESH,
    scratch_types=[
        pltpu.VMEM((NCH_W, CH), jnp.int32),
        pltpu.VMEM((NCH_W, CH), jnp.int32),
        pltpu.VMEM((CH, OUT), jnp.float32),
        pltpu.VMEM((CH, OUT), jnp.float32),
        pltpu.VMEM((E // NW,), jnp.float32),
        pltpu.SemaphoreType.DMA,
        pltpu.SemaphoreType.DMA,
    ],
)
def _decode_kernel(z_hbm, src_hbm, dst_hbm, out_hbm,
                   src_v, dst_v, zs_v, zd_v, lg_v, sem_s, sem_d):
    c = lax.axis_index("c")
    s = lax.axis_index("s")
    wid = s * NC + c
    pltpu.sync_copy(src_hbm.at[wid], src_v)
    pltpu.sync_copy(dst_hbm.at[wid], dst_v)
    lanes = lax.iota(jnp.int32, 16)

    def chunk(j, carry):
        cs = pltpu.async_copy(z_hbm.at[src_v.at[j]], zs_v, sem_s)
        cd = pltpu.async_copy(z_hbm.at[dst_v.at[j]], zd_v, sem_d)
        cs.wait()
        cd.wait()
        for g in range(CH // 16):
            rows = lanes + (g * 16)

            def dot_step(k, acc):
                col = jnp.full((16,), k, dtype=jnp.int32)
                a = plsc.load_gather(zs_v, [rows, col])
                b = plsc.load_gather(zd_v, [rows, col])
                return acc + a * b

            acc = lax.fori_loop(0, OUT, dot_step, jnp.zeros((16,), jnp.float32))
            lg_v[pl.ds(j * CH + g * 16, 16)] = acc
        return carry

    lax.fori_loop(0, NCH_W, chunk, 0)
    pltpu.sync_copy(lg_v, out_hbm.at[pl.ds(wid * (E // NW), E // NW)])


def _prep_body(degacc_ref, x_ref, w1_ref, dinv_ref, xw1p_ref):
    d = degacc_ref[0] + degacc_ref[1]
    deg = jnp.sum(d, axis=1, keepdims=True) + 1.0
    dinv = 1.0 / jnp.sqrt(deg)
    dinv_ref[...] = dinv
    xw = jnp.dot(x_ref[...], w1_ref[...], preferred_element_type=jnp.float32)
    xw1p_ref[...] = xw * dinv


def _prep_call(degacc, x_pad, w1):
    return pl.pallas_call(
        _prep_body,
        out_shape=[
            jax.ShapeDtypeStruct((N_PAD, 1), jnp.float32),
            jax.ShapeDtypeStruct((N_PAD, HID), jnp.float32),
        ],
    )(degacc, x_pad, w1)


def _mid_body(acc_ref, xw1p_ref, dinv_ref, b1_ref, g1_ref, be1_ref, w2_ref,
              hwp_ref):
    dinv = dinv_ref[...]
    t = (acc_ref[0] + acc_ref[1] + xw1p_ref[...]) * dinv + b1_ref[...]
    rows = lax.broadcasted_iota(jnp.int32, (N_PAD, 1), 0)
    m = (rows < N).astype(jnp.float32)
    cnt = jnp.float32(N)
    mu = jnp.sum(t * m, axis=0, keepdims=True) / cnt
    var = jnp.sum((t - mu) * (t - mu) * m, axis=0, keepdims=True) / cnt
    h = (t - mu) / jnp.sqrt(var + EPS) * g1_ref[...] + be1_ref[...]
    h = jnp.maximum(h, 0.0)
    hw = jnp.dot(h, w2_ref[...], preferred_element_type=jnp.float32)
    hwp = hw * dinv
    hwp_ref[0, :, :] = hwp[:, :128]
    hwp_ref[1, :, :] = hwp[:, 128:]


def _mid_call(acc1, xw1p, dinv, b1, g1, be1, w2):
    return pl.pallas_call(
        _mid_body,
        out_shape=jax.ShapeDtypeStruct((NC, N_PAD, 128), jnp.float32),
    )(acc1, xw1p, dinv, b1, g1, be1, w2)


def _final_body(acc2_ref, hwp_ref, dinv_ref, b2_ref, z_ref):
    dinv = dinv_ref[...]
    z_ref[:, :128] = (acc2_ref[0] + hwp_ref[0]) * dinv + b2_ref[:, :128]
    z_ref[:, 128:] = (acc2_ref[1] + hwp_ref[1]) * dinv + b2_ref[:, 128:]


def _final_call(acc2, hwp, dinv, b2):
    return pl.pallas_call(
        _final_body,
        out_shape=jax.ShapeDtypeStruct((N_PAD, OUT), jnp.float32),
    )(acc2, hwp, dinv, b2)


def kernel(x, edge_index, W1, b1, gamma1, beta1, W2, b2):
    src = edge_index[0]
    dst = edge_index[1]
    x_pad = jnp.pad(x, ((0, N_PAD - N), (0, 0)))
    src_w = src.reshape(NW, NCH_W, CH)
    dst_w = dst.reshape(NW, NCH_W, CH)
    src_s = src.reshape(NS, NCH_S, CH)
    dst_s = dst.reshape(NS, NCH_S, CH)
    ones8 = jnp.ones((CH, 8), jnp.float32)
    zeros8 = jnp.zeros((N_PAD, 8), jnp.float32)
    zeros64 = jnp.zeros((N_PAD, HID), jnp.float32)
    zeros128 = jnp.zeros((N_PAD, 128), jnp.float32)

    degacc = _deg_kernel(dst_w, ones8, zeros8)
    dinv, xw1p = _prep_call(degacc, x_pad, W1)
    acc1 = _agg64_kernel(xw1p, src_w, dst_w, zeros64)
    hwp = _mid_call(acc1, xw1p, dinv, b1.reshape(1, HID),
                    gamma1.reshape(1, HID), beta1.reshape(1, HID), W2)
    acc2 = _agg128_kernel(hwp, src_s, dst_s, zeros128)
    z = _final_call(acc2, hwp, dinv, b2.reshape(1, OUT))
    return _decode_kernel(z, src_w, dst_w)


# speedup vs baseline: 3.6975x; 3.6975x over previous
import functools

import jax
import jax.numpy as jnp
from jax import lax
from jax.experimental import pallas as pl
from jax.experimental.pallas import tpu as pltpu
from jax.experimental.pallas import tpu_sc as plsc

N = 10000
E = 320000
IN_DIM = 128
HID = 64
OUT = 256
EPS = 1e-5

NC = 2
NS = 16
NW = NC * NS
CH = 80
NCH_W = E // (NW * CH)
NCH_S = E // (NS * CH)
SEG = 25
N_PAD = 10112
RPT = N_PAD // NS

_MESH = plsc.VectorSubcoreMesh(
    core_axis_name="c", subcore_axis_name="s", num_cores=NC, num_subcores=NS)


@functools.partial(
    pl.kernel,
    out_type=jax.ShapeDtypeStruct((NC, N_PAD, 128), jnp.float32),
    mesh=_MESH,
    scratch_types=[
        pltpu.VMEM((NCH_W, CH), jnp.int32),
        pltpu.VMEM((CH, 128), jnp.float32),
        pltpu.VMEM_SHARED((N_PAD, 128), jnp.float32),
        pltpu.SemaphoreType.DMA,
    ],
)
def _deg_kernel(dst_hbm, ones_hbm, zeros_hbm, out_hbm, dst_v, ones_v, acc_sh, sem):
    c = lax.axis_index("c")
    s = lax.axis_index("s")
    wid = s * NC + c
    r0 = s * RPT
    pltpu.sync_copy(dst_hbm.at[wid], dst_v)
    pltpu.sync_copy(ones_hbm, ones_v)
    pltpu.sync_copy(zeros_hbm.at[pl.ds(r0, RPT)], acc_sh.at[pl.ds(r0, RPT)])
    plsc.subcore_barrier()

    def body(j, carry):
        pltpu.sync_copy(ones_v, acc_sh.at[dst_v.at[j]], add=True)
        return carry

    lax.fori_loop(0, NCH_W, body, 0)
    plsc.subcore_barrier()
    pltpu.sync_copy(acc_sh.at[pl.ds(r0, RPT)], out_hbm.at[c, pl.ds(r0, RPT)])


@functools.partial(
    pl.kernel,
    out_type=jax.ShapeDtypeStruct((NC, N_PAD, 128), jnp.float32),
    mesh=_MESH,
    scratch_types=[
        pltpu.VMEM((SEG, CH), jnp.int32),
        pltpu.VMEM((SEG, CH), jnp.int32),
        pltpu.VMEM((CH, 128), jnp.float32),
        pltpu.VMEM_SHARED((N_PAD, 128), jnp.float32),
        pltpu.SemaphoreType.DMA,
    ],
)
def _agg64_kernel(table_hbm, src_hbm, dst_hbm, zeros_hbm, out_hbm,
                  src_v, dst_v, rows_v, acc_sh, sem):
    c = lax.axis_index("c")
    s = lax.axis_index("s")
    wid = s * NC + c
    r0 = s * RPT
    pltpu.sync_copy(zeros_hbm.at[pl.ds(r0, RPT)], acc_sh.at[pl.ds(r0, RPT)])
    plsc.subcore_barrier()

    def body(j, carry):
        pltpu.async_copy(table_hbm.at[src_v.at[j]], rows_v, sem).wait()
        pltpu.sync_copy(rows_v, acc_sh.at[dst_v.at[j]], add=True)
        return carry

    for g in range(NCH_W // SEG):
        pltpu.sync_copy(src_hbm.at[wid, g], src_v)
        pltpu.sync_copy(dst_hbm.at[wid, g], dst_v)
        lax.fori_loop(0, SEG, body, 0)
    plsc.subcore_barrier()
    pltpu.sync_copy(acc_sh.at[pl.ds(r0, RPT)], out_hbm.at[c, pl.ds(r0, RPT)])


@functools.partial(
    pl.kernel,
    out_type=jax.ShapeDtypeStruct((NC, N_PAD, 128), jnp.float32),
    mesh=_MESH,
    scratch_types=[
        pltpu.VMEM((SEG, CH), jnp.int32),
        pltpu.VMEM((SEG, CH), jnp.int32),
        pltpu.VMEM((CH, 128), jnp.float32),
        pltpu.VMEM_SHARED((N_PAD, 128), jnp.float32),
        pltpu.SemaphoreType.DMA,
    ],
)
def _agg128_kernel(t0_hbm, t1_hbm, src_hbm, dst_hbm, zeros_hbm, out_hbm,
                   src_v, dst_v, rows_v, acc_sh, sem):
    c = lax.axis_index("c")
    s = lax.axis_index("s")
    r0 = s * RPT
    pltpu.sync_copy(zeros_hbm.at[pl.ds(r0, RPT)], acc_sh.at[pl.ds(r0, RPT)])
    plsc.subcore_barrier()

    def body(j, carry):
        @pl.when(c == 0)
        def _():
            pltpu.async_copy(t0_hbm.at[src_v.at[j]], rows_v, sem).wait()

        @pl.when(c == 1)
        def _():
            pltpu.async_copy(t1_hbm.at[src_v.at[j]], rows_v, sem).wait()

        pltpu.sync_copy(rows_v, acc_sh.at[dst_v.at[j]], add=True)
        return carry

    for g in range(NCH_S // SEG):
        pltpu.sync_copy(src_hbm.at[s, g], src_v)
        pltpu.sync_copy(dst_hbm.at[s, g], dst_v)
        lax.fori_loop(0, SEG, body, 0)
    plsc.subcore_barrier()
    pltpu.sync_copy(acc_sh.at[pl.ds(r0, RPT)], out_hbm.at[c, pl.ds(r0, RPT)])


@functools.partial(
    pl.kernel,
    out_type=jax.ShapeDtypeStruct((E,), jnp.float32),
    mesh=_MESH,
    scratch_types=[
        pltpu.VMEM((NCH_W, CH), jnp.int32),
        pltpu.VMEM((NCH_W, CH), jnp.int32),
        pltpu.VMEM((CH, OUT), jnp.float32),
        pltpu.VMEM((CH, OUT), jnp.float32),
        pltpu.VMEM((E // NW,), jnp.float32),
        pltpu.SemaphoreType.DMA,
        pltpu.SemaphoreType.DMA,
    ],
    compiler_params=pltpu.CompilerParams(needs_layout_passes=False),
)
def _decode_kernel(z_hbm, src_hbm, dst_hbm, out_hbm,
                   src_v, dst_v, zs_v, zd_v, lg_v, sem_s, sem_d):
    c = lax.axis_index("c")
    s = lax.axis_index("s")
    wid = s * NC + c
    pltpu.sync_copy(src_hbm.at[wid], src_v)
    pltpu.sync_copy(dst_hbm.at[wid], dst_v)
    lanes = lax.iota(jnp.int32, 16)

    def chunk(j, carry):
        cs = pltpu.async_copy(z_hbm.at[src_v.at[j]], zs_v, sem_s)
        cd = pltpu.async_copy(z_hbm.at[dst_v.at[j]], zd_v, sem_d)
        cs.wait()
        cd.wait()
        for g in range(CH // 16):
            rows = lanes + (g * 16)

            def dot_step(k, acc):
                col = jnp.full((16,), k, dtype=jnp.int32)
                a = plsc.load_gather(zs_v, [rows, col])
                b = plsc.load_gather(zd_v, [rows, col])
                return acc + a * b

            acc = lax.fori_loop(0, OUT, dot_step, jnp.zeros((16,), jnp.float32))
            lg_v[pl.ds(j * CH + g * 16, 16)] = acc
        return carry

    lax.fori_loop(0, NCH_W, chunk, 0)
    pltpu.sync_copy(lg_v, out_hbm.at[pl.ds(wid * (E // NW), E // NW)])


def _prep_body(degacc_ref, x_ref, w1_ref, dinv_ref, xw1p_ref):
    deg = degacc_ref[0, :, 0:1] + degacc_ref[1, :, 0:1] + 1.0
    dinv = 1.0 / jnp.sqrt(deg)
    dinv_ref[...] = dinv
    xw = jnp.dot(x_ref[...], w1_ref[...], preferred_element_type=jnp.float32)
    xw1p_ref[:, :HID] = xw * dinv
    xw1p_ref[:, HID:] = jnp.zeros((N_PAD, 128 - HID), jnp.float32)


def _prep_call(degacc, x_pad, w1):
    return pl.pallas_call(
        _prep_body,
        out_shape=[
            jax.ShapeDtypeStruct((N_PAD, 1), jnp.float32),
            jax.ShapeDtypeStruct((N_PAD, 128), jnp.float32),
        ],
    )(degacc, x_pad, w1)


def _mid_body(acc_ref, xw1p_ref, dinv_ref, b1_ref, g1_ref, be1_ref, w2_ref,
              lo_ref, hi_ref):
    dinv = dinv_ref[...]
    t = (acc_ref[0, :, :HID] + acc_ref[1, :, :HID] + xw1p_ref[:, :HID]) * dinv \
        + b1_ref[...]
    rows = lax.broadcasted_iota(jnp.int32, (N_PAD, 1), 0)
    m = (rows < N).astype(jnp.float32)
    cnt = jnp.float32(N)
    mu = jnp.sum(t * m, axis=0, keepdims=True) / cnt
    var = jnp.sum((t - mu) * (t - mu) * m, axis=0, keepdims=True) / cnt
    h = (t - mu) / jnp.sqrt(var + EPS) * g1_ref[...] + be1_ref[...]
    h = jnp.maximum(h, 0.0)
    hw = jnp.dot(h, w2_ref[...], preferred_element_type=jnp.float32)
    hwp = hw * dinv
    lo_ref[...] = hwp[:, :128]
    hi_ref[...] = hwp[:, 128:]


def _mid_call(acc1, xw1p, dinv, b1, g1, be1, w2):
    return pl.pallas_call(
        _mid_body,
        out_shape=[
            jax.ShapeDtypeStruct((N_PAD, 128), jnp.float32),
            jax.ShapeDtypeStruct((N_PAD, 128), jnp.float32),
        ],
    )(acc1, xw1p, dinv, b1, g1, be1, w2)


def _final_body(acc2_ref, lo_ref, hi_ref, dinv_ref, b2_ref, z_ref):
    dinv = dinv_ref[...]
    z_ref[:, :128] = (acc2_ref[0] + lo_ref[...]) * dinv + b2_ref[:, :128]
    z_ref[:, 128:] = (acc2_ref[1] + hi_ref[...]) * dinv + b2_ref[:, 128:]


def _final_call(acc2, hwp_lo, hwp_hi, dinv, b2):
    return pl.pallas_call(
        _final_body,
        out_shape=jax.ShapeDtypeStruct((N_PAD, OUT), jnp.float32),
    )(acc2, hwp_lo, hwp_hi, dinv, b2)


def kernel(x, edge_index, W1, b1, gamma1, beta1, W2, b2):
    src = edge_index[0]
    dst = edge_index[1]
    x_pad = jnp.pad(x, ((0, N_PAD - N), (0, 0)))
    src_w = src.reshape(NW, NCH_W, CH)
    dst_w = dst.reshape(NW, NCH_W, CH)
    src_w4 = src.reshape(NW, NCH_W // SEG, SEG, CH)
    dst_w4 = dst.reshape(NW, NCH_W // SEG, SEG, CH)
    src_s4 = src.reshape(NS, NCH_S // SEG, SEG, CH)
    dst_s4 = dst.reshape(NS, NCH_S // SEG, SEG, CH)
    ones128 = jnp.ones((CH, 128), jnp.float32)
    zeros128 = jnp.zeros((N_PAD, 128), jnp.float32)

    degacc = _deg_kernel(dst_w, ones128, zeros128)
    dinv, xw1p = _prep_call(degacc, x_pad, W1)
    acc1 = _agg64_kernel(xw1p, src_w4, dst_w4, zeros128)
    hwp_lo, hwp_hi = _mid_call(acc1, xw1p, dinv, b1.reshape(1, HID),
                               gamma1.reshape(1, HID), beta1.reshape(1, HID),
                               W2)
    acc2 = _agg128_kernel(hwp_lo, hwp_hi, src_s4, dst_s4, zeros128)
    z = _final_call(acc2, hwp_lo, hwp_hi, dinv, b2.reshape(1, OUT))
    return _decode_kernel(z, src_w, dst_w)


# speedup vs baseline: 4.1521x; 1.1229x over previous
import functools

import jax
import jax.numpy as jnp
from jax import lax
from jax.experimental import pallas as pl
from jax.experimental.pallas import tpu as pltpu
from jax.experimental.pallas import tpu_sc as plsc

N = 10000
E = 320000
IN_DIM = 128
HID = 64
OUT = 256
EPS = 1e-5

NC = 2
NS = 16
NW = NC * NS
CH = 80
NCH_W = E // (NW * CH)
NCH_S = E // (NS * CH)
SEG = 25
N_PAD = 10112
RPT = N_PAD // NS

_MESH = plsc.VectorSubcoreMesh(
    core_axis_name="c", subcore_axis_name="s", num_cores=NC, num_subcores=NS)


@functools.partial(
    pl.kernel,
    out_type=jax.ShapeDtypeStruct((NC, N_PAD, 128), jnp.float32),
    mesh=_MESH,
    scratch_types=[
        pltpu.VMEM((NCH_W, CH), jnp.int32),
        pltpu.VMEM((CH, 128), jnp.float32),
        pltpu.VMEM_SHARED((N_PAD, 128), jnp.float32),
        pltpu.SemaphoreType.DMA,
    ],
)
def _deg_kernel(dst_hbm, ones_hbm, zeros_hbm, out_hbm, dst_v, ones_v, acc_sh, sem):
    c = lax.axis_index("c")
    s = lax.axis_index("s")
    wid = s * NC + c
    r0 = s * RPT
    pltpu.sync_copy(dst_hbm.at[wid], dst_v)
    pltpu.sync_copy(ones_hbm, ones_v)
    pltpu.sync_copy(zeros_hbm.at[pl.ds(r0, RPT)], acc_sh.at[pl.ds(r0, RPT)])
    plsc.subcore_barrier()

    def body(j, carry):
        pltpu.sync_copy(ones_v, acc_sh.at[dst_v.at[j]], add=True)
        return carry

    lax.fori_loop(0, NCH_W, body, 0)
    plsc.subcore_barrier()
    pltpu.sync_copy(acc_sh.at[pl.ds(r0, RPT)], out_hbm.at[c, pl.ds(r0, RPT)])


@functools.partial(
    pl.kernel,
    out_type=jax.ShapeDtypeStruct((NC, N_PAD, 128), jnp.float32),
    mesh=_MESH,
    scratch_types=[
        pltpu.VMEM((SEG, CH), jnp.int32),
        pltpu.VMEM((SEG, CH), jnp.int32),
        pltpu.VMEM((2, CH, 128), jnp.float32),
        pltpu.VMEM_SHARED((N_PAD, 128), jnp.float32),
        pltpu.SemaphoreType.DMA,
    ],
)
def _agg64_kernel(table_hbm, src_hbm, dst_hbm, zeros_hbm, out_hbm,
                  src_v, dst_v, rows_v, acc_sh, sem):
    c = lax.axis_index("c")
    s = lax.axis_index("s")
    wid = s * NC + c
    r0 = s * RPT
    pltpu.sync_copy(zeros_hbm.at[pl.ds(r0, RPT)], acc_sh.at[pl.ds(r0, RPT)])
    plsc.subcore_barrier()

    def gather(jj, slot):
        pltpu.async_copy(table_hbm.at[src_v.at[jj]], rows_v.at[slot], sem)

    def gwait(slot):
        pltpu.make_async_copy(
            table_hbm.at[src_v.at[0]], rows_v.at[slot], sem).wait()

    def scatter(jj, slot):
        pltpu.sync_copy(rows_v.at[slot], acc_sh.at[dst_v.at[jj]], add=True)

    for g in range(NCH_W // SEG):
        pltpu.sync_copy(src_hbm.at[wid, g], src_v)
        pltpu.sync_copy(dst_hbm.at[wid, g], dst_v)
        gather(0, 0)
        gather(1, 1)

        def pair(t, carry):
            j0 = t * 2
            gwait(0)
            scatter(j0, 0)
            gather(j0 + 2, 0)
            gwait(1)
            scatter(j0 + 1, 1)

            @pl.when(j0 + 3 < SEG)
            def _():
                gather(j0 + 3, 1)

            return carry

        lax.fori_loop(0, (SEG - 1) // 2, pair, 0)
        gwait(0)
        scatter(SEG - 1, 0)
    plsc.subcore_barrier()
    pltpu.sync_copy(acc_sh.at[pl.ds(r0, RPT)], out_hbm.at[c, pl.ds(r0, RPT)])


@functools.partial(
    pl.kernel,
    out_type=jax.ShapeDtypeStruct((NC, N_PAD, 128), jnp.float32),
    mesh=_MESH,
    scratch_types=[
        pltpu.VMEM((SEG, CH), jnp.int32),
        pltpu.VMEM((SEG, CH), jnp.int32),
        pltpu.VMEM((2, CH, 128), jnp.float32),
        pltpu.VMEM_SHARED((N_PAD, 128), jnp.float32),
        pltpu.SemaphoreType.DMA,
    ],
)
def _agg128_kernel(t0_hbm, t1_hbm, src_hbm, dst_hbm, zeros_hbm, out_hbm,
                   src_v, dst_v, rows_v, acc_sh, sem):
    c = lax.axis_index("c")
    s = lax.axis_index("s")
    r0 = s * RPT
    pltpu.sync_copy(zeros_hbm.at[pl.ds(r0, RPT)], acc_sh.at[pl.ds(r0, RPT)])
    plsc.subcore_barrier()

    def gather(jj, slot):
        @pl.when(c == 0)
        def _():
            pltpu.async_copy(t0_hbm.at[src_v.at[jj]], rows_v.at[slot], sem)

        @pl.when(c == 1)
        def _():
            pltpu.async_copy(t1_hbm.at[src_v.at[jj]], rows_v.at[slot], sem)

    def gwait(slot):
        pltpu.make_async_copy(
            t0_hbm.at[src_v.at[0]], rows_v.at[slot], sem).wait()

    def scatter(jj, slot):
        pltpu.sync_copy(rows_v.at[slot], acc_sh.at[dst_v.at[jj]], add=True)

    for g in range(NCH_S // SEG):
        pltpu.sync_copy(src_hbm.at[s, g], src_v)
        pltpu.sync_copy(dst_hbm.at[s, g], dst_v)
        gather(0, 0)
        gather(1, 1)

        def pair(t, carry):
            j0 = t * 2
            gwait(0)
            scatter(j0, 0)
            gather(j0 + 2, 0)
            gwait(1)
            scatter(j0 + 1, 1)

            @pl.when(j0 + 3 < SEG)
            def _():
                gather(j0 + 3, 1)

            return carry

        lax.fori_loop(0, (SEG - 1) // 2, pair, 0)
        gwait(0)
        scatter(SEG - 1, 0)
    plsc.subcore_barrier()
    pltpu.sync_copy(acc_sh.at[pl.ds(r0, RPT)], out_hbm.at[c, pl.ds(r0, RPT)])


@functools.partial(
    pl.kernel,
    out_type=jax.ShapeDtypeStruct((E,), jnp.float32),
    mesh=_MESH,
    scratch_types=[
        pltpu.VMEM((SEG, CH), jnp.int32),
        pltpu.VMEM((SEG, CH), jnp.int32),
        pltpu.VMEM((2, CH, OUT), jnp.float32),
        pltpu.VMEM((2, CH, OUT), jnp.float32),
        pltpu.VMEM((E // NW,), jnp.float32),
        pltpu.SemaphoreType.DMA,
        pltpu.SemaphoreType.DMA,
    ],
    compiler_params=pltpu.CompilerParams(needs_layout_passes=False),
)
def _decode_kernel(z_hbm, src_hbm, dst_hbm, out_hbm,
                   src_v, dst_v, zs_v, zd_v, lg_v, sem_s, sem_d):
    c = lax.axis_index("c")
    s = lax.axis_index("s")
    wid = s * NC + c
    lanes = lax.iota(jnp.int32, 16)
    z16 = jnp.zeros((16,), jnp.float32)

    def issue(jj, slot):
        pltpu.async_copy(z_hbm.at[src_v.at[jj]], zs_v.at[slot], sem_s)
        pltpu.async_copy(z_hbm.at[dst_v.at[jj]], zd_v.at[slot], sem_d)

    def dwait(slot):
        pltpu.make_async_copy(z_hbm.at[src_v.at[0]], zs_v.at[slot], sem_s).wait()
        pltpu.make_async_copy(z_hbm.at[dst_v.at[0]], zd_v.at[slot], sem_d).wait()

    def compute(cbase, slot):
        zs = zs_v.at[slot]
        zd = zd_v.at[slot]
        for gg in range(CH // 16):
            rows = lanes + gg * 16

            def dot_step(kk, accs):
                accs = list(accs)
                k0 = kk * 8
                for u in range(8):
                    col = jnp.full((16,), k0 + u, dtype=jnp.int32)
                    a = plsc.load_gather(zs, [rows, col])
                    b = plsc.load_gather(zd, [rows, col])
                    accs[u % 4] = accs[u % 4] + a * b
                return tuple(accs)

            a0, a1, a2, a3 = lax.fori_loop(
                0, OUT // 8, dot_step, (z16, z16, z16, z16))
            lg_v[pl.ds(cbase * CH + gg * 16, 16)] = (a0 + a1) + (a2 + a3)

    for g in range(NCH_W // SEG):
        pltpu.sync_copy(src_hbm.at[wid, g], src_v)
        pltpu.sync_copy(dst_hbm.at[wid, g], dst_v)
        issue(0, 0)
        issue(1, 1)

        def pair(t, carry):
            j0 = t * 2
            dwait(0)
            compute(g * SEG + j0, 0)
            issue(j0 + 2, 0)
            dwait(1)
            compute(g * SEG + j0 + 1, 1)

            @pl.when(j0 + 3 < SEG)
            def _():
                issue(j0 + 3, 1)

            return carry

        lax.fori_loop(0, (SEG - 1) // 2, pair, 0)
        dwait(0)
        compute(g * SEG + SEG - 1, 0)
    pltpu.sync_copy(lg_v, out_hbm.at[pl.ds(wid * (E // NW), E // NW)])


def _prep_body(degacc_ref, x_ref, w1_ref, dinv_ref, xw1p_ref):
    deg = degacc_ref[0, :, 0:1] + degacc_ref[1, :, 0:1] + 1.0
    dinv = 1.0 / jnp.sqrt(deg)
    dinv_ref[...] = dinv
    xw = jnp.dot(x_ref[...], w1_ref[...], preferred_element_type=jnp.float32)
    xw1p_ref[:, :HID] = xw * dinv
    xw1p_ref[:, HID:] = jnp.zeros((N_PAD, 128 - HID), jnp.float32)


def _prep_call(degacc, x_pad, w1):
    return pl.pallas_call(
        _prep_body,
        out_shape=[
            jax.ShapeDtypeStruct((N_PAD, 1), jnp.float32),
            jax.ShapeDtypeStruct((N_PAD, 128), jnp.float32),
        ],
    )(degacc, x_pad, w1)


def _mid_body(acc_ref, xw1p_ref, dinv_ref, b1_ref, g1_ref, be1_ref, w2_ref,
              lo_ref, hi_ref):
    dinv = dinv_ref[...]
    t = (acc_ref[0, :, :HID] + acc_ref[1, :, :HID] + xw1p_ref[:, :HID]) * dinv \
        + b1_ref[...]
    rows = lax.broadcasted_iota(jnp.int32, (N_PAD, 1), 0)
    m = (rows < N).astype(jnp.float32)
    cnt = jnp.float32(N)
    mu = jnp.sum(t * m, axis=0, keepdims=True) / cnt
    var = jnp.sum((t - mu) * (t - mu) * m, axis=0, keepdims=True) / cnt
    h = (t - mu) / jnp.sqrt(var + EPS) * g1_ref[...] + be1_ref[...]
    h = jnp.maximum(h, 0.0)
    hw = jnp.dot(h, w2_ref[...], preferred_element_type=jnp.float32)
    hwp = hw * dinv
    lo_ref[...] = hwp[:, :128]
    hi_ref[...] = hwp[:, 128:]


def _mid_call(acc1, xw1p, dinv, b1, g1, be1, w2):
    return pl.pallas_call(
        _mid_body,
        out_shape=[
            jax.ShapeDtypeStruct((N_PAD, 128), jnp.float32),
            jax.ShapeDtypeStruct((N_PAD, 128), jnp.float32),
        ],
    )(acc1, xw1p, dinv, b1, g1, be1, w2)


def _final_body(acc2_ref, lo_ref, hi_ref, dinv_ref, b2_ref, z_ref):
    dinv = dinv_ref[...]
    z_ref[:, :128] = (acc2_ref[0] + lo_ref[...]) * dinv + b2_ref[:, :128]
    z_ref[:, 128:] = (acc2_ref[1] + hi_ref[...]) * dinv + b2_ref[:, 128:]


def _final_call(acc2, hwp_lo, hwp_hi, dinv, b2):
    return pl.pallas_call(
        _final_body,
        out_shape=jax.ShapeDtypeStruct((N_PAD, OUT), jnp.float32),
    )(acc2, hwp_lo, hwp_hi, dinv, b2)


def kernel(x, edge_index, W1, b1, gamma1, beta1, W2, b2):
    src = edge_index[0]
    dst = edge_index[1]
    x_pad = jnp.pad(x, ((0, N_PAD - N), (0, 0)))
    dst_w = dst.reshape(NW, NCH_W, CH)
    src_w4 = src.reshape(NW, NCH_W // SEG, SEG, CH)
    dst_w4 = dst.reshape(NW, NCH_W // SEG, SEG, CH)
    src_s4 = src.reshape(NS, NCH_S // SEG, SEG, CH)
    dst_s4 = dst.reshape(NS, NCH_S // SEG, SEG, CH)
    ones128 = jnp.ones((CH, 128), jnp.float32)
    zeros128 = jnp.zeros((N_PAD, 128), jnp.float32)

    degacc = _deg_kernel(dst_w, ones128, zeros128)
    dinv, xw1p = _prep_call(degacc, x_pad, W1)
    acc1 = _agg64_kernel(xw1p, src_w4, dst_w4, zeros128)
    hwp_lo, hwp_hi = _mid_call(acc1, xw1p, dinv, b1.reshape(1, HID),
                               gamma1.reshape(1, HID), beta1.reshape(1, HID),
                               W2)
    acc2 = _agg128_kernel(hwp_lo, hwp_hi, src_s4, dst_s4, zeros128)
    z = _final_call(acc2, hwp_lo, hwp_hi, dinv, b2.reshape(1, OUT))
    return _decode_kernel(z, src_w4, dst_w4)


# speedup vs baseline: 18.2831x; 4.4034x over previous
import functools

import jax
import jax.numpy as jnp
from jax import lax
from jax.experimental import pallas as pl
from jax.experimental.pallas import tpu as pltpu
from jax.experimental.pallas import tpu_sc as plsc

N = 10000
E = 320000
IN_DIM = 128
HID = 64
OUT = 256
EPS = 1e-5

NC = 2
NS = 16
NW = NC * NS
CH = 80
NCH_W = E // (NW * CH)
NCH_S = E // (NS * CH)
SEG = 25
N_PAD = 10112
RPT = N_PAD // NS
CH2 = 128
NCH2 = 160
SEG2 = 20
E2_TILE = NCH2 * CH2
TRASH = N_PAD - 1

_MESH = plsc.VectorSubcoreMesh(
    core_axis_name="c", subcore_axis_name="s", num_cores=NC, num_subcores=NS)


@functools.partial(
    pl.kernel,
    out_type=jax.ShapeDtypeStruct((NW, N_PAD), jnp.float32),
    mesh=_MESH,
    scratch_types=[
        pltpu.VMEM((E // NW,), jnp.int32),
        pltpu.VMEM((N_PAD,), jnp.float32),
    ],
    compiler_params=pltpu.CompilerParams(needs_layout_passes=False),
)
def _deg_kernel(dst_hbm, out_hbm, dst_v, cnt_v):
    c = lax.axis_index("c")
    s = lax.axis_index("s")
    wid = s * NC + c
    pltpu.sync_copy(dst_hbm.at[wid], dst_v)
    z16 = jnp.zeros((16,), jnp.float32)
    ones = jnp.ones((16,), jnp.float32)

    def zbody(i, carry):
        cnt_v[pl.ds(i * 16, 16)] = z16
        return carry

    lax.fori_loop(0, N_PAD // 16, zbody, 0)

    def body(i, carry):
        idx = dst_v[pl.ds(i * 16, 16)]
        plsc.addupdate_scatter(cnt_v, [idx], ones)
        return carry

    lax.fori_loop(0, E // (NW * 16), body, 0)
    pltpu.sync_copy(cnt_v, out_hbm.at[wid])


@functools.partial(
    pl.kernel,
    out_type=jax.ShapeDtypeStruct((NC, N_PAD, HID), jnp.float32),
    mesh=_MESH,
    scratch_types=[
        pltpu.VMEM((SEG, CH), jnp.int32),
        pltpu.VMEM((SEG, CH), jnp.int32),
        pltpu.VMEM((2, CH, HID), jnp.float32),
        pltpu.VMEM_SHARED((N_PAD, HID), jnp.float32),
        pltpu.SemaphoreType.DMA,
    ],
    compiler_params=pltpu.CompilerParams(use_tc_tiling_on_sc=False),
)
def _agg64_kernel(table_hbm, src_hbm, dst_hbm, zeros_hbm, out_hbm,
                  src_v, dst_v, rows_v, acc_sh, sem):
    c = lax.axis_index("c")
    s = lax.axis_index("s")
    wid = s * NC + c
    r0 = s * RPT
    pltpu.sync_copy(zeros_hbm.at[pl.ds(r0, RPT)], acc_sh.at[pl.ds(r0, RPT)])
    plsc.subcore_barrier()

    def gather(jj, slot):
        pltpu.async_copy(table_hbm.at[src_v.at[jj]], rows_v.at[slot], sem)

    def gwait(slot):
        pltpu.make_async_copy(
            table_hbm.at[src_v.at[0]], rows_v.at[slot], sem).wait()

    def scatter(jj, slot):
        pltpu.sync_copy(rows_v.at[slot], acc_sh.at[dst_v.at[jj]], add=True)

    for g in range(NCH_W // SEG):
        pltpu.sync_copy(src_hbm.at[wid, g], src_v)
        pltpu.sync_copy(dst_hbm.at[wid, g], dst_v)
        gather(0, 0)
        gather(1, 1)

        def pair(t, carry):
            j0 = t * 2
            gwait(0)
            scatter(j0, 0)
            gather(j0 + 2, 0)
            gwait(1)
            scatter(j0 + 1, 1)

            @pl.when(j0 + 3 < SEG)
            def _():
                gather(j0 + 3, 1)

            return carry

        lax.fori_loop(0, (SEG - 1) // 2, pair, 0)
        gwait(0)
        scatter(SEG - 1, 0)
    plsc.subcore_barrier()
    pltpu.sync_copy(acc_sh.at[pl.ds(r0, RPT)], out_hbm.at[c, pl.ds(r0, RPT)])


@functools.partial(
    pl.kernel,
    out_type=jax.ShapeDtypeStruct((NC, N_PAD, 128), jnp.float32),
    mesh=_MESH,
    scratch_types=[
        pltpu.VMEM((SEG, CH), jnp.int32),
        pltpu.VMEM((SEG, CH), jnp.int32),
        pltpu.VMEM((2, CH, 128), jnp.float32),
        pltpu.VMEM_SHARED((N_PAD, 128), jnp.float32),
        pltpu.SemaphoreType.DMA,
    ],
)
def _agg128_kernel(t0_hbm, t1_hbm, src_hbm, dst_hbm, zeros_hbm, out_hbm,
                   src_v, dst_v, rows_v, acc_sh, sem):
    c = lax.axis_index("c")
    s = lax.axis_index("s")
    r0 = s * RPT
    pltpu.sync_copy(zeros_hbm.at[pl.ds(r0, RPT)], acc_sh.at[pl.ds(r0, RPT)])
    plsc.subcore_barrier()

    def gather(jj, slot):
        @pl.when(c == 0)
        def _():
            pltpu.async_copy(t0_hbm.at[src_v.at[jj]], rows_v.at[slot], sem)

        @pl.when(c == 1)
        def _():
            pltpu.async_copy(t1_hbm.at[src_v.at[jj]], rows_v.at[slot], sem)

    def gwait(slot):
        pltpu.make_async_copy(
            t0_hbm.at[src_v.at[0]], rows_v.at[slot], sem).wait()

    def scatter(jj, slot):
        pltpu.sync_copy(rows_v.at[slot], acc_sh.at[dst_v.at[jj]], add=True)

    for g in range(NCH_S // SEG):
        pltpu.sync_copy(src_hbm.at[s, g], src_v)
        pltpu.sync_copy(dst_hbm.at[s, g], dst_v)
        gather(0, 0)
        gather(1, 1)

        def pair(t, carry):
            j0 = t * 2
            gwait(0)
            scatter(j0, 0)
            gather(j0 + 2, 0)
            gwait(1)
            scatter(j0 + 1, 1)

            @pl.when(j0 + 3 < SEG)
            def _():
                gather(j0 + 3, 1)

            return carry

        lax.fori_loop(0, (SEG - 1) // 2, pair, 0)
        gwait(0)
        scatter(SEG - 1, 0)
    plsc.subcore_barrier()
    pltpu.sync_copy(acc_sh.at[pl.ds(r0, RPT)], out_hbm.at[c, pl.ds(r0, RPT)])


@functools.partial(
    pl.kernel,
    out_type=jax.ShapeDtypeStruct((E,), jnp.float32),
    mesh=_MESH,
    scratch_types=[
        pltpu.VMEM((SEG, CH), jnp.int32),
        pltpu.VMEM((SEG, CH), jnp.int32),
        pltpu.VMEM((2, CH, OUT), jnp.float32),
        pltpu.VMEM((2, CH, OUT), jnp.float32),
        pltpu.VMEM((E // NW,), jnp.float32),
        pltpu.SemaphoreType.DMA,
        pltpu.SemaphoreType.DMA,
    ],
    compiler_params=pltpu.CompilerParams(needs_layout_passes=False),
)
def _decode_kernel(z_hbm, src_hbm, dst_hbm, out_hbm,
                   src_v, dst_v, zs_v, zd_v, lg_v, sem_s, sem_d):
    c = lax.axis_index("c")
    s = lax.axis_index("s")
    wid = s * NC + c
    lanes = lax.iota(jnp.int32, 16)
    z16 = jnp.zeros((16,), jnp.float32)

    def issue(jj, slot):
        pltpu.async_copy(z_hbm.at[src_v.at[jj]], zs_v.at[slot], sem_s)
        pltpu.async_copy(z_hbm.at[dst_v.at[jj]], zd_v.at[slot], sem_d)

    def dwait(slot):
        pltpu.make_async_copy(z_hbm.at[src_v.at[0]], zs_v.at[slot], sem_s).wait()
        pltpu.make_async_copy(z_hbm.at[dst_v.at[0]], zd_v.at[slot], sem_d).wait()

    perms = [jnp.bitwise_xor(lanes, d) for d in (8, 4, 2, 1)]

    def compute(cbase, slot):
        zs = zs_v.at[slot]
        zd = zd_v.at[slot]
        for gg in range(CH // 16):

            def edge_body(t, res):
                e = gg * 16 + t
                accs = [z16, z16, z16, z16]
                for k in range(OUT // 16):
                    a = zs[e, pl.ds(k * 16, 16)]
                    b = zd[e, pl.ds(k * 16, 16)]
                    accs[k % 4] = accs[k % 4] + a * b
                acc = (accs[0] + accs[1]) + (accs[2] + accs[3])
                for p in perms:
                    acc = acc + acc[p]
                return jnp.where(lanes == t, acc, res)

            res = lax.fori_loop(0, 16, edge_body, z16)
            lg_v[pl.ds(cbase * CH + gg * 16, 16)] = res

    def segment(g, carry):
        pltpu.sync_copy(src_hbm.at[wid, g], src_v)
        pltpu.sync_copy(dst_hbm.at[wid, g], dst_v)
        issue(0, 0)
        issue(1, 1)

        def pair(t, carry2):
            j0 = t * 2
            dwait(0)
            compute(g * SEG + j0, 0)
            issue(j0 + 2, 0)
            dwait(1)
            compute(g * SEG + j0 + 1, 1)

            @pl.when(j0 + 3 < SEG)
            def _():
                issue(j0 + 3, 1)

            return carry2

        lax.fori_loop(0, (SEG - 1) // 2, pair, 0)
        dwait(0)
        compute(g * SEG + SEG - 1, 0)
        return carry

    lax.fori_loop(0, NCH_W // SEG, segment, 0)
    pltpu.sync_copy(lg_v, out_hbm.at[pl.ds(wid * (E // NW), E // NW)])


def _prep_body(degacc_ref, x_ref, w1_ref, dinv_ref, xw1p_ref):
    deg = jnp.sum(degacc_ref[...], axis=0)[:, None] + 1.0
    dinv = 1.0 / jnp.sqrt(deg)
    dinv_ref[...] = dinv
    xw = jnp.dot(x_ref[...], w1_ref[...], preferred_element_type=jnp.float32)
    xw1p_ref[...] = xw * dinv


def _prep_call(degacc, x_pad, w1):
    return pl.pallas_call(
        _prep_body,
        out_shape=[
            jax.ShapeDtypeStruct((N_PAD, 1), jnp.float32),
            jax.ShapeDtypeStruct((N_PAD, HID), jnp.float32),
        ],
    )(degacc, x_pad, w1)


def _mid_body(acc_ref, xw1p_ref, dinv_ref, b1_ref, g1_ref, be1_ref, w2_ref,
              lo_ref, hi_ref):
    dinv = dinv_ref[...]
    t = (acc_ref[0] + acc_ref[1] + xw1p_ref[...]) * dinv + b1_ref[...]
    rows = lax.broadcasted_iota(jnp.int32, (N_PAD, 1), 0)
    m = (rows < N).astype(jnp.float32)
    cnt = jnp.float32(N)
    mu = jnp.sum(t * m, axis=0, keepdims=True) / cnt
    var = jnp.sum((t - mu) * (t - mu) * m, axis=0, keepdims=True) / cnt
    h = (t - mu) / jnp.sqrt(var + EPS) * g1_ref[...] + be1_ref[...]
    h = jnp.maximum(h, 0.0)
    hw = jnp.dot(h, w2_ref[...], preferred_element_type=jnp.float32)
    hwp = hw * dinv
    lo_ref[...] = hwp[:, :128]
    hi_ref[...] = hwp[:, 128:]


def _mid_call(acc1, xw1p, dinv, b1, g1, be1, w2):
    return pl.pallas_call(
        _mid_body,
        out_shape=[
            jax.ShapeDtypeStruct((N_PAD, 128), jnp.float32),
            jax.ShapeDtypeStruct((N_PAD, 128), jnp.float32),
        ],
    )(acc1, xw1p, dinv, b1, g1, be1, w2)


def _final_body(acc2_ref, lo_ref, hi_ref, dinv_ref, b2_ref, z_ref):
    dinv = dinv_ref[...]
    z_ref[:, :128] = (acc2_ref[0] + lo_ref[...]) * dinv + b2_ref[:, :128]
    z_ref[:, 128:] = (acc2_ref[1] + hi_ref[...]) * dinv + b2_ref[:, 128:]


def _final_call(acc2, hwp_lo, hwp_hi, dinv, b2):
    return pl.pallas_call(
        _final_body,
        out_shape=jax.ShapeDtypeStruct((N_PAD, OUT), jnp.float32),
    )(acc2, hwp_lo, hwp_hi, dinv, b2)


def kernel(x, edge_index, W1, b1, gamma1, beta1, W2, b2):
    src = edge_index[0]
    dst = edge_index[1]
    x_pad = jnp.pad(x, ((0, N_PAD - N), (0, 0)))
    src_w4 = src.reshape(NW, NCH_W // SEG, SEG, CH)
    dst_w4 = dst.reshape(NW, NCH_W // SEG, SEG, CH)
    src_s4 = src.reshape(NS, NCH_S // SEG, SEG, CH)
    dst_s4 = dst.reshape(NS, NCH_S // SEG, SEG, CH)
    zeros64 = jnp.zeros((N_PAD, HID), jnp.float32)
    zeros128 = jnp.zeros((N_PAD, 128), jnp.float32)

    degacc = _deg_kernel(dst.reshape(NW, E // NW))
    dinv, xw1p = _prep_call(degacc, x_pad, W1)
    acc1 = _agg64_kernel(xw1p, src_w4, dst_w4, zeros64)
    hwp_lo, hwp_hi = _mid_call(acc1, xw1p, dinv, b1.reshape(1, HID),
                               gamma1.reshape(1, HID), beta1.reshape(1, HID),
                               W2)
    acc2 = _agg128_kernel(hwp_lo, hwp_hi, src_s4, dst_s4, zeros128)
    z = _final_call(acc2, hwp_lo, hwp_hi, dinv, b2.reshape(1, OUT))
    return _decode_kernel(z, src_w4, dst_w4)


# speedup vs baseline: 18.3296x; 1.0025x over previous
import functools

import jax
import jax.numpy as jnp
from jax import lax
from jax.experimental import pallas as pl
from jax.experimental.pallas import tpu as pltpu
from jax.experimental.pallas import tpu_sc as plsc

N = 10000
E = 320000
IN_DIM = 128
HID = 64
OUT = 256
EPS = 1e-5

NC = 2
NS = 16
NW = NC * NS
CH = 80
NCH_W = E // (NW * CH)
NCH_S = E // (NS * CH)
SEG = 25
N_PAD = 10112
RPT = N_PAD // NS

_MESH = plsc.VectorSubcoreMesh(
    core_axis_name="c", subcore_axis_name="s", num_cores=NC, num_subcores=NS)


@functools.partial(
    pl.kernel,
    out_type=jax.ShapeDtypeStruct((NW, N_PAD), jnp.float32),
    mesh=_MESH,
    scratch_types=[
        pltpu.VMEM((E // NW,), jnp.int32),
        pltpu.VMEM((N_PAD,), jnp.float32),
    ],
    compiler_params=pltpu.CompilerParams(needs_layout_passes=False),
)
def _deg_kernel(dst_hbm, out_hbm, dst_v, cnt_v):
    c = lax.axis_index("c")
    s = lax.axis_index("s")
    wid = s * NC + c
    pltpu.sync_copy(dst_hbm.at[wid], dst_v)
    z16 = jnp.zeros((16,), jnp.float32)
    ones = jnp.ones((16,), jnp.float32)

    def zbody(i, carry):
        cnt_v[pl.ds(i * 16, 16)] = z16
        return carry

    lax.fori_loop(0, N_PAD // 16, zbody, 0)

    def body(i, carry):
        idx = dst_v[pl.ds(i * 16, 16)]
        plsc.addupdate_scatter(cnt_v, [idx], ones)
        return carry

    lax.fori_loop(0, E // (NW * 16), body, 0)
    pltpu.sync_copy(cnt_v, out_hbm.at[wid])


@functools.partial(
    pl.kernel,
    out_type=jax.ShapeDtypeStruct((NC, N_PAD, HID), jnp.float32),
    mesh=_MESH,
    scratch_types=[
        pltpu.VMEM((SEG, CH), jnp.int32),
        pltpu.VMEM((SEG, CH), jnp.int32),
        pltpu.VMEM((2, CH, HID), jnp.float32),
        pltpu.VMEM_SHARED((N_PAD, HID), jnp.float32),
        pltpu.SemaphoreType.DMA,
    ],
    compiler_params=pltpu.CompilerParams(use_tc_tiling_on_sc=False),
)
def _agg64_kernel(table_hbm, src_hbm, dst_hbm, zeros_hbm, out_hbm,
                  src_v, dst_v, rows_v, acc_sh, sem):
    c = lax.axis_index("c")
    s = lax.axis_index("s")
    wid = s * NC + c
    r0 = s * RPT
    pltpu.sync_copy(zeros_hbm.at[pl.ds(r0, RPT)], acc_sh.at[pl.ds(r0, RPT)])
    plsc.subcore_barrier()

    def gather(jj, slot):
        pltpu.async_copy(table_hbm.at[src_v.at[jj]], rows_v.at[slot], sem)

    def gwait(slot):
        pltpu.make_async_copy(
            table_hbm.at[src_v.at[0]], rows_v.at[slot], sem).wait()

    def scatter(jj, slot):
        pltpu.sync_copy(rows_v.at[slot], acc_sh.at[dst_v.at[jj]], add=True)

    for g in range(NCH_W // SEG):
        pltpu.sync_copy(src_hbm.at[wid, g], src_v)
        pltpu.sync_copy(dst_hbm.at[wid, g], dst_v)
        gather(0, 0)
        gather(1, 1)

        def pair(t, carry):
            j0 = t * 2
            gwait(0)
            scatter(j0, 0)
            gather(j0 + 2, 0)
            gwait(1)
            scatter(j0 + 1, 1)

            @pl.when(j0 + 3 < SEG)
            def _():
                gather(j0 + 3, 1)

            return carry

        lax.fori_loop(0, (SEG - 1) // 2, pair, 0)
        gwait(0)
        scatter(SEG - 1, 0)
    plsc.subcore_barrier()
    pltpu.sync_copy(acc_sh.at[pl.ds(r0, RPT)], out_hbm.at[c, pl.ds(r0, RPT)])


@functools.partial(
    pl.kernel,
    out_type=jax.ShapeDtypeStruct((NC, N_PAD, 128), jnp.float32),
    mesh=_MESH,
    scratch_types=[
        pltpu.VMEM((SEG, CH), jnp.int32),
        pltpu.VMEM((SEG, CH), jnp.int32),
        pltpu.VMEM((2, CH, 128), jnp.float32),
        pltpu.VMEM_SHARED((N_PAD, 128), jnp.float32),
        pltpu.SemaphoreType.DMA,
    ],
)
def _agg128_kernel(t0_hbm, t1_hbm, src_hbm, dst_hbm, zeros_hbm, out_hbm,
                   src_v, dst_v, rows_v, acc_sh, sem):
    c = lax.axis_index("c")
    s = lax.axis_index("s")
    r0 = s * RPT
    pltpu.sync_copy(zeros_hbm.at[pl.ds(r0, RPT)], acc_sh.at[pl.ds(r0, RPT)])
    plsc.subcore_barrier()

    def gather(jj, slot):
        @pl.when(c == 0)
        def _():
            pltpu.async_copy(t0_hbm.at[src_v.at[jj]], rows_v.at[slot], sem)

        @pl.when(c == 1)
        def _():
            pltpu.async_copy(t1_hbm.at[src_v.at[jj]], rows_v.at[slot], sem)

    def gwait(slot):
        pltpu.make_async_copy(
            t0_hbm.at[src_v.at[0]], rows_v.at[slot], sem).wait()

    def scatter(jj, slot):
        pltpu.sync_copy(rows_v.at[slot], acc_sh.at[dst_v.at[jj]], add=True)

    for g in range(NCH_S // SEG):
        pltpu.sync_copy(src_hbm.at[s, g], src_v)
        pltpu.sync_copy(dst_hbm.at[s, g], dst_v)
        gather(0, 0)
        gather(1, 1)

        def pair(t, carry):
            j0 = t * 2
            gwait(0)
            scatter(j0, 0)
            gather(j0 + 2, 0)
            gwait(1)
            scatter(j0 + 1, 1)

            @pl.when(j0 + 3 < SEG)
            def _():
                gather(j0 + 3, 1)

            return carry

        lax.fori_loop(0, (SEG - 1) // 2, pair, 0)
        gwait(0)
        scatter(SEG - 1, 0)
    plsc.subcore_barrier()
    pltpu.sync_copy(acc_sh.at[pl.ds(r0, RPT)], out_hbm.at[c, pl.ds(r0, RPT)])


@functools.partial(
    pl.kernel,
    out_type=jax.ShapeDtypeStruct((E,), jnp.float32),
    mesh=_MESH,
    scratch_types=[
        pltpu.VMEM((SEG, CH), jnp.int32),
        pltpu.VMEM((SEG, CH), jnp.int32),
        pltpu.VMEM((2, CH, OUT), jnp.float32),
        pltpu.VMEM((2, CH, OUT), jnp.float32),
        pltpu.VMEM((E // NW,), jnp.float32),
        pltpu.SemaphoreType.DMA,
        pltpu.SemaphoreType.DMA,
    ],
    compiler_params=pltpu.CompilerParams(needs_layout_passes=False),
)
def _decode_kernel(z_hbm, src_hbm, dst_hbm, out_hbm,
                   src_v, dst_v, zs_v, zd_v, lg_v, sem_s, sem_d):
    c = lax.axis_index("c")
    s = lax.axis_index("s")
    wid = s * NC + c
    lanes = lax.iota(jnp.int32, 16)
    z16 = jnp.zeros((16,), jnp.float32)

    def issue(jj, slot):
        pltpu.async_copy(z_hbm.at[src_v.at[jj]], zs_v.at[slot], sem_s)
        pltpu.async_copy(z_hbm.at[dst_v.at[jj]], zd_v.at[slot], sem_d)

    def dwait(slot):
        pltpu.make_async_copy(z_hbm.at[src_v.at[0]], zs_v.at[slot], sem_s).wait()
        pltpu.make_async_copy(z_hbm.at[dst_v.at[0]], zd_v.at[slot], sem_d).wait()

    perms = [jnp.bitwise_xor(lanes, d) for d in (8, 4, 2, 1)]

    def compute(cbase, slot):
        zs = zs_v.at[slot]
        zd = zd_v.at[slot]
        for gg in range(CH // 16):

            def edge_body(t, res):
                e = gg * 16 + t
                accs = [z16, z16, z16, z16]
                for k in range(OUT // 16):
                    a = zs[e, pl.ds(k * 16, 16)]
                    b = zd[e, pl.ds(k * 16, 16)]
                    accs[k % 4] = accs[k % 4] + a * b
                acc = (accs[0] + accs[1]) + (accs[2] + accs[3])
                for p in perms:
                    acc = acc + acc[p]
                return jnp.where(lanes == t, acc, res)

            res = lax.fori_loop(0, 16, edge_body, z16)
            lg_v[pl.ds(cbase * CH + gg * 16, 16)] = res

    def segment(g, carry):
        pltpu.sync_copy(src_hbm.at[wid, g], src_v)
        pltpu.sync_copy(dst_hbm.at[wid, g], dst_v)
        issue(0, 0)
        issue(1, 1)

        def pair(t, carry2):
            j0 = t * 2
            dwait(0)
            compute(g * SEG + j0, 0)
            issue(j0 + 2, 0)
            dwait(1)
            compute(g * SEG + j0 + 1, 1)

            @pl.when(j0 + 3 < SEG)
            def _():
                issue(j0 + 3, 1)

            return carry2

        lax.fori_loop(0, (SEG - 1) // 2, pair, 0)
        dwait(0)
        compute(g * SEG + SEG - 1, 0)
        return carry

    lax.fori_loop(0, NCH_W // SEG, segment, 0)
    pltpu.sync_copy(lg_v, out_hbm.at[pl.ds(wid * (E // NW), E // NW)])


def _prep_body(degacc_ref, x_ref, w1_ref, dinv_ref, xw1p_ref):
    deg = jnp.sum(degacc_ref[...], axis=0)[:, None] + 1.0
    dinv = 1.0 / jnp.sqrt(deg)
    dinv_ref[...] = dinv
    xw = jnp.dot(x_ref[...], w1_ref[...], preferred_element_type=jnp.float32)
    xw1p_ref[...] = xw * dinv


def _prep_call(degacc, x_pad, w1):
    return pl.pallas_call(
        _prep_body,
        out_shape=[
            jax.ShapeDtypeStruct((N_PAD, 1), jnp.float32),
            jax.ShapeDtypeStruct((N_PAD, HID), jnp.float32),
        ],
    )(degacc, x_pad, w1)


def _mid_body(acc_ref, xw1p_ref, dinv_ref, b1_ref, g1_ref, be1_ref, w2_ref,
              lo_ref, hi_ref):
    dinv = dinv_ref[...]
    t = (acc_ref[0] + acc_ref[1] + xw1p_ref[...]) * dinv + b1_ref[...]
    rows = lax.broadcasted_iota(jnp.int32, (N_PAD, 1), 0)
    m = (rows < N).astype(jnp.float32)
    cnt = jnp.float32(N)
    mu = jnp.sum(t * m, axis=0, keepdims=True) / cnt
    var = jnp.sum((t - mu) * (t - mu) * m, axis=0, keepdims=True) / cnt
    h = (t - mu) / jnp.sqrt(var + EPS) * g1_ref[...] + be1_ref[...]
    h = jnp.maximum(h, 0.0)
    hw = jnp.dot(h, w2_ref[...], preferred_element_type=jnp.float32)
    hwp = hw * dinv
    lo_ref[...] = hwp[:, :128]
    hi_ref[...] = hwp[:, 128:]


def _mid_call(acc1, xw1p, dinv, b1, g1, be1, w2):
    return pl.pallas_call(
        _mid_body,
        out_shape=[
            jax.ShapeDtypeStruct((N_PAD, 128), jnp.float32),
            jax.ShapeDtypeStruct((N_PAD, 128), jnp.float32),
        ],
    )(acc1, xw1p, dinv, b1, g1, be1, w2)


def _final_body(acc2_ref, lo_ref, hi_ref, dinv_ref, b2_ref, z_ref):
    dinv = dinv_ref[...]
    z_ref[:, :128] = (acc2_ref[0] + lo_ref[...]) * dinv + b2_ref[:, :128]
    z_ref[:, 128:] = (acc2_ref[1] + hi_ref[...]) * dinv + b2_ref[:, 128:]


def _final_call(acc2, hwp_lo, hwp_hi, dinv, b2):
    return pl.pallas_call(
        _final_body,
        out_shape=jax.ShapeDtypeStruct((N_PAD, OUT), jnp.float32),
    )(acc2, hwp_lo, hwp_hi, dinv, b2)


def kernel(x, edge_index, W1, b1, gamma1, beta1, W2, b2):
    src = edge_index[0]
    dst = edge_index[1]
    x_pad = jnp.pad(x, ((0, N_PAD - N), (0, 0)))
    src_w4 = src.reshape(NW, NCH_W // SEG, SEG, CH)
    dst_w4 = dst.reshape(NW, NCH_W // SEG, SEG, CH)
    src_s4 = src.reshape(NS, NCH_S // SEG, SEG, CH)
    dst_s4 = dst.reshape(NS, NCH_S // SEG, SEG, CH)
    zeros64 = jnp.zeros((N_PAD, HID), jnp.float32)
    zeros128 = jnp.zeros((N_PAD, 128), jnp.float32)

    degacc = _deg_kernel(dst.reshape(NW, E // NW))
    dinv, xw1p = _prep_call(degacc, x_pad, W1)
    acc1 = _agg64_kernel(xw1p, src_w4, dst_w4, zeros64)
    hwp_lo, hwp_hi = _mid_call(acc1, xw1p, dinv, b1.reshape(1, HID),
                               gamma1.reshape(1, HID), beta1.reshape(1, HID),
                               W2)
    acc2 = _agg128_kernel(hwp_lo, hwp_hi, src_s4, dst_s4, zeros128)
    z = _final_call(acc2, hwp_lo, hwp_hi, dinv, b2.reshape(1, OUT))
    return _decode_kernel(z, src_w4, dst_w4)
